# Initial kernel scaffold; baseline (speedup 1.0000x reference)
import jax
import jax.numpy as jnp
from jax.experimental import pallas as pl


def kernel(x, gso, W0, b0, W1, b1):
    raise NotImplementedError("write your pallas kernel here")


# speedup vs baseline: 1.1383x; 1.1383x over previous
import jax
import jax.numpy as jnp
from jax.experimental import pallas as pl

_BM = 400


def _pass1_body(gso_ref, x_ref, xb_ref, w0_ref, b0_ref, gbf_ref, p0_ref, r0_ref):
    g = gso_ref[...].astype(jnp.bfloat16)
    gbf_ref[...] = g
    y0 = jnp.dot(g, x_ref[...], preferred_element_type=jnp.float32)
    y0b = y0.astype(jnp.bfloat16)
    w0 = w0_ref[...]
    p0_ref[...] = jnp.dot(y0b, w0[2], preferred_element_type=jnp.float32).astype(
        jnp.bfloat16
    )
    r0_ref[...] = (
        jnp.dot(xb_ref[...], w0[0] - w0[2], preferred_element_type=jnp.float32)
        + jnp.dot(y0b, w0[1], preferred_element_type=jnp.float32)
        + b0_ref[...]
    )


def _pass2_body(gbf_ref, p0_ref, r0_ref, w1_ref, b1_ref, hbf_ref, r1_ref):
    out0 = (
        2.0 * jnp.dot(gbf_ref[...], p0_ref[...], preferred_element_type=jnp.float32)
        + r0_ref[...]
    )
    hb = jnp.maximum(out0, 0.0).astype(jnp.bfloat16)
    hbf_ref[...] = hb
    w1 = w1_ref[...]
    r1_ref[...] = (
        jnp.dot(hb, w1[0] - w1[2], preferred_element_type=jnp.float32) + b1_ref[...]
    )


def _pass3_body(gbf_ref, hbf_ref, r1_ref, w1_ref, q1_ref, s1_ref):
    y1 = jnp.dot(gbf_ref[...], hbf_ref[...], preferred_element_type=jnp.float32)
    y1b = y1.astype(jnp.bfloat16)
    w1 = w1_ref[...]
    q1_ref[...] = jnp.dot(y1b, w1[2], preferred_element_type=jnp.float32).astype(
        jnp.bfloat16
    )
    s1_ref[...] = r1_ref[...] + jnp.dot(
        y1b, w1[1], preferred_element_type=jnp.float32
    )


def _pass4_body(n_class, gbf_ref, q1_ref, s1_ref, out_ref):
    logits = (
        2.0 * jnp.dot(gbf_ref[...], q1_ref[...], preferred_element_type=jnp.float32)
        + s1_ref[...]
    )
    mask = jax.lax.broadcasted_iota(jnp.int32, logits.shape, 1) < n_class
    ml = jnp.where(mask, logits, -jnp.inf)
    m = jnp.max(ml, axis=1, keepdims=True)
    e = jnp.where(mask, jnp.exp(ml - m), 0.0)
    lse = m + jnp.log(jnp.sum(e, axis=1, keepdims=True))
    out_ref[...] = logits - lse


def kernel(x, gso, W0, b0, W1, b1):
    n, d = x.shape
    _, _, h_dim = W0.shape
    c = W1.shape[2]
    cp = 128
    nblk = n // _BM

    xb16 = x.astype(jnp.bfloat16)
    w0b = W0.astype(jnp.bfloat16)
    w1b = jnp.zeros((W1.shape[0], h_dim, cp), jnp.bfloat16)
    w1b = w1b.at[:, :, :c].set(W1.astype(jnp.bfloat16))
    b0r = b0.reshape(1, h_dim)
    b1r = jnp.zeros((1, cp), jnp.float32).at[0, :c].set(b1)

    row_blk = lambda bs: pl.BlockSpec(bs, lambda i: (i, 0))
    full2 = lambda shape: pl.BlockSpec(shape, lambda i: (0, 0))

    gbf, p0, r0 = pl.pallas_call(
        _pass1_body,
        grid=(nblk,),
        in_specs=[
            row_blk((_BM, n)),
            full2((n, d)),
            row_blk((_BM, d)),
            pl.BlockSpec((W0.shape[0], d, h_dim), lambda i: (0, 0, 0)),
            full2((1, h_dim)),
        ],
        out_specs=[
            row_blk((_BM, n)),
            row_blk((_BM, h_dim)),
            row_blk((_BM, h_dim)),
        ],
        out_shape=[
            jax.ShapeDtypeStruct((n, n), jnp.bfloat16),
            jax.ShapeDtypeStruct((n, h_dim), jnp.bfloat16),
            jax.ShapeDtypeStruct((n, h_dim), jnp.float32),
        ],
    )(gso, xb16, xb16, w0b, b0r)

    hbf, r1 = pl.pallas_call(
        _pass2_body,
        grid=(nblk,),
        in_specs=[
            row_blk((_BM, n)),
            full2((n, h_dim)),
            row_blk((_BM, h_dim)),
            pl.BlockSpec((W1.shape[0], h_dim, cp), lambda i: (0, 0, 0)),
            full2((1, cp)),
        ],
        out_specs=[row_blk((_BM, h_dim)), row_blk((_BM, cp))],
        out_shape=[
            jax.ShapeDtypeStruct((n, h_dim), jnp.bfloat16),
            jax.ShapeDtypeStruct((n, cp), jnp.float32),
        ],
    )(gbf, p0, r0, w1b, b1r)

    q1, s1 = pl.pallas_call(
        _pass3_body,
        grid=(nblk,),
        in_specs=[
            row_blk((_BM, n)),
            full2((n, h_dim)),
            row_blk((_BM, cp)),
            pl.BlockSpec((W1.shape[0], h_dim, cp), lambda i: (0, 0, 0)),
        ],
        out_specs=[row_blk((_BM, cp)), row_blk((_BM, cp))],
        out_shape=[
            jax.ShapeDtypeStruct((n, cp), jnp.bfloat16),
            jax.ShapeDtypeStruct((n, cp), jnp.float32),
        ],
    )(gbf, hbf, r1, w1b)

    out_pad = pl.pallas_call(
        lambda *refs: _pass4_body(c, *refs),
        grid=(nblk,),
        in_specs=[
            row_blk((_BM, n)),
            full2((n, cp)),
            row_blk((_BM, cp)),
        ],
        out_specs=row_blk((_BM, cp)),
        out_shape=jax.ShapeDtypeStruct((n, cp), jnp.float32),
    )(gbf, q1, s1)

    return out_pad[:, :c]
